# Initial kernel scaffold; baseline (speedup 1.0000x reference)
import jax
import jax.numpy as jnp
from jax.experimental import pallas as pl


def kernel(hidden_states, WK, bK, WV, bV, ReadingHead, bidirection_weight):
    raise NotImplementedError("write your pallas kernel here")


# speedup vs baseline: 1920.6283x; 1920.6283x over previous
import functools

import jax
import jax.numpy as jnp
from jax import lax
from jax.experimental import pallas as pl
from jax.experimental.pallas import tpu as pltpu
from jax.experimental.pallas import tpu_sc as plsc

_NH = 12
_HD = 64
_HID = 768
_NR = 8
_L = 2048
_BS = 2
_LB = 512
_CH = 32
_NCHUNK = _L // _CH
_NPAIR = _BS * _NH


def _tc_body(h_ref, wkt_ref, bk_ref, wvt_ref, bv_ref, rh_ref, seg_ref,
             v_ref, m_ref):
    x = h_ref[0]
    v = jnp.dot(x, wvt_ref[...], preferred_element_type=jnp.float32,
                precision=lax.Precision.DEFAULT) + bv_ref[0]
    v_ref[0] = jnp.maximum(v, 0.0)
    k = jnp.dot(x, wkt_ref[...], preferred_element_type=jnp.float32,
                precision=lax.Precision.DEFAULT) + bk_ref[0]
    k = jnp.maximum(k, 0.0)
    k16 = k.astype(jnp.bfloat16).astype(jnp.float32)
    rh16 = rh_ref[0].astype(jnp.bfloat16).astype(jnp.float32)
    dot = jnp.dot(k16 * rh16, seg_ref[...],
                  preferred_element_type=jnp.float32,
                  precision=lax.Precision.HIGHEST)
    m_ref[0] = (dot > 0.5).astype(jnp.int32)


def _sc_body(vrows, maskh, wh, outh, maskblk, cum, p_ref, wrow, ilist, rows,
             outc, gsem):
    wid = lax.axis_index("s") * 2 + lax.axis_index("c")

    @pl.when(wid < _NPAIR)
    def _():
        b = wid // _NH
        h = wid % _NH
        pltpu.sync_copy(maskh.at[b], maskblk)
        pltpu.sync_copy(wh.at[h], wrow)
        lanes = lax.iota(jnp.int32, 16)

        def scan_step(v, c):
            base = v * 16
            pos = lanes + base
            mvec = plsc.load_gather(maskblk, [pos * _NH + h])
            mvec = jnp.where(pos == 0, 0, mvec)
            cum[pl.ds(base, 16)] = plsc.cumsum(mvec) + c
            plsc.store_scatter(p_ref, [jnp.maximum(cum[pl.ds(base, 16)] - 1, 0)],
                               pos, mask=mvec != 0)
            return c + jnp.sum(mvec)

        cnt = lax.fori_loop(0, _L // 16, scan_step, jnp.int32(0))

        wvec = wrow[...]
        ws = [wvec[r] for r in range(2 * _NR)]
        vbase = (b * _L) * _NH + h
        outbase = (b * _NH + h) * _L

        subrow = jnp.where(lanes >= 8, 1, 0)
        colbase = (lanes % 8) * 16

        def chunk_step(cidx, _):
            for v in range(_CH // 16):
                pos = lanes + cidx * _CH + v * 16
                cumv = cum[pl.ds(cidx * _CH + v * 16, 16)]
                cprev = plsc.load_gather(cum, [jnp.maximum(pos - 1, 0)])
                cprev = jnp.where(pos > 0, cprev, 0)
                rowv = subrow + 2 * v
                for r in range(_NR):
                    fi = cumv - (1 + r)
                    fj = plsc.load_gather(p_ref, [jnp.maximum(fi, 0)])
                    fj = jnp.where(fi >= 0, fj, 0)
                    plsc.store_scatter(ilist, [rowv, colbase + r],
                                       fj * _NH + vbase)
                    bi = cprev + r
                    bj = plsc.load_gather(p_ref, [bi])
                    bj = jnp.where(jnp.logical_and(bi < cnt, pos > 0), bj, 0)
                    plsc.store_scatter(ilist, [rowv, colbase + r + _NR],
                                       bj * _NH + vbase)
            cps = [pltpu.async_copy(vrows.at[ilist.at[j]], rows.at[j], gsem)
                   for j in range(4)]
            for cp in cps:
                cp.wait()

            def acc_body(i, _):
                ri = rows.at[i // 8]
                m = (i % 8) * 16
                oi = outc.at[i]
                for k in range(_HD // 16):
                    a = ws[0] * ri[m, pl.ds(k * 16, 16)]
                    for r in range(1, 2 * _NR):
                        a = a + ws[r] * ri[m + r, pl.ds(k * 16, 16)]
                    oi[pl.ds(k * 16, 16)] = a
                return 0

            lax.fori_loop(0, _CH, acc_body, 0)
            pltpu.sync_copy(outc, outh.at[pl.ds(outbase + cidx * _CH, _CH)])
            return 0

        lax.fori_loop(0, _NCHUNK, chunk_step, 0)


_SC_KERNEL = functools.partial(
    pl.kernel,
    out_type=jax.ShapeDtypeStruct((_BS * _NH * _L, _HD), jnp.float32),
    mesh=plsc.VectorSubcoreMesh(core_axis_name="c", subcore_axis_name="s",
                                num_cores=2, num_subcores=16),
    compiler_params=pltpu.CompilerParams(needs_layout_passes=False,
                                         use_tc_tiling_on_sc=False),
    scratch_types=[
        pltpu.VMEM((_L * _NH,), jnp.int32),
        pltpu.VMEM((_L,), jnp.int32),
        pltpu.VMEM((_L + 16,), jnp.int32),
        pltpu.VMEM((2 * _NR,), jnp.float32),
        pltpu.VMEM((4, 128), jnp.int32),
        pltpu.VMEM((4, 128, _HD), jnp.float32),
        pltpu.VMEM((_CH, _HD), jnp.float32),
        pltpu.SemaphoreType.DMA,
    ],
)(_sc_body)


def kernel(hidden_states, WK, bK, WV, bV, ReadingHead, bidirection_weight):
    wkt = WK.T
    wvt = WV.T
    seg = jnp.repeat(jnp.eye(_NH, dtype=jnp.float32), _HD, axis=0)
    rh = ReadingHead.reshape(1, _NH * _HD)
    bk = bK.reshape(1, _HID)
    bv = bV.reshape(1, _HID)

    V, M = pl.pallas_call(
        _tc_body,
        grid=(_BS, _L // _LB),
        in_specs=[
            pl.BlockSpec((1, _LB, _HID), lambda b, l: (b, l, 0)),
            pl.BlockSpec((_HID, _HID), lambda b, l: (0, 0)),
            pl.BlockSpec((1, _HID), lambda b, l: (0, 0)),
            pl.BlockSpec((_HID, _HID), lambda b, l: (0, 0)),
            pl.BlockSpec((1, _HID), lambda b, l: (0, 0)),
            pl.BlockSpec((1, _HID), lambda b, l: (0, 0)),
            pl.BlockSpec((_HID, _NH), lambda b, l: (0, 0)),
        ],
        out_specs=[
            pl.BlockSpec((1, _LB, _HID), lambda b, l: (b, l, 0)),
            pl.BlockSpec((1, _LB, _NH), lambda b, l: (b, l, 0)),
        ],
        out_shape=[
            jax.ShapeDtypeStruct((_BS, _L, _HID), jnp.float32),
            jax.ShapeDtypeStruct((_BS, _L, _NH), jnp.int32),
        ],
    )(hidden_states, wkt, bk, wvt, bv, rh, seg)

    vrows = V.reshape(_BS * _L * _NH, _HD)
    mask2 = M.reshape(_BS, _L * _NH)
    wmat = bidirection_weight.reshape(_NH, 2 * _NR)
    outr = _SC_KERNEL(vrows, mask2, wmat)
    return (outr.reshape(_BS, _NH, _L, _HD)
            .transpose(0, 2, 1, 3)
            .reshape(_BS, _L, _NH * _HD))


# speedup vs baseline: 2504.7711x; 1.3041x over previous
import functools

import jax
import jax.numpy as jnp
from jax import lax
from jax.experimental import pallas as pl
from jax.experimental.pallas import tpu as pltpu
from jax.experimental.pallas import tpu_sc as plsc

_NH = 12
_HD = 64
_HID = 768
_NR = 8
_L = 2048
_BS = 2
_LB = 512
_CH = 32
_NCHUNK = _L // _CH
_NPAIR = _BS * _NH


def _tc_body(h_ref, wkt_ref, bk_ref, wvt_ref, bv_ref, rh_ref, seg_ref,
             v_ref, m_ref):
    x = h_ref[0]
    v = jnp.dot(x, wvt_ref[...], preferred_element_type=jnp.float32,
                precision=lax.Precision.DEFAULT) + bv_ref[0]
    v_ref[0] = jnp.maximum(v, 0.0)
    k = jnp.dot(x, wkt_ref[...], preferred_element_type=jnp.float32,
                precision=lax.Precision.DEFAULT) + bk_ref[0]
    k = jnp.maximum(k, 0.0)
    k16 = k.astype(jnp.bfloat16).astype(jnp.float32)
    rh16 = rh_ref[0].astype(jnp.bfloat16).astype(jnp.float32)
    dot = jnp.dot(k16 * rh16, seg_ref[...],
                  preferred_element_type=jnp.float32,
                  precision=lax.Precision.HIGHEST)
    m_ref[0] = (dot > 0.5).astype(jnp.int32)


_UPT = 3
_CPU_ = _NCHUNK // 4


def _sc_body(vrows, maskh, wh, outh, maskblk, cum, p_ref, wrow,
             il0, il1, rows0, rows1, outc, sem0, sem1):
    wid = lax.axis_index("s") * 2 + lax.axis_index("c")
    lanes = lax.iota(jnp.int32, 16)
    subrow = jnp.where(lanes >= 8, 1, 0)
    colbase = (lanes % 8) * 16

    for s in range(_UPT):
        u = wid * _UPT + s
        p = u // 4
        q = u % 4
        b = p // _NH
        h = p % _NH
        pltpu.sync_copy(maskh.at[b], maskblk)
        pltpu.sync_copy(wh.at[h], wrow)

        def scan_step(v, c, h=h):
            base = v * 16
            pos = lanes + base
            mvec = plsc.load_gather(maskblk, [pos * _NH + h])
            mvec = jnp.where(pos == 0, 0, mvec)
            cum[pl.ds(base, 16)] = plsc.cumsum(mvec) + c
            plsc.store_scatter(p_ref, [jnp.maximum(cum[pl.ds(base, 16)] - 1, 0)],
                               pos, mask=mvec != 0)
            return c + jnp.sum(mvec)

        cnt = lax.fori_loop(0, _L // 16, scan_step, jnp.int32(0))

        wvec = wrow[...]
        ws = [wvec[r] for r in range(2 * _NR)]
        vbase = (b * _L) * _NH + h
        outbase = (b * _NH + h) * _L

        def build(cidx, ilist, cnt=cnt, vbase=vbase):
            for v in range(_CH // 16):
                pos = lanes + cidx * _CH + v * 16
                cumv = cum[pl.ds(cidx * _CH + v * 16, 16)]
                cprev = plsc.load_gather(cum, [jnp.maximum(pos - 1, 0)])
                cprev = jnp.where(pos > 0, cprev, 0)
                rowv = subrow + 2 * v
                for r in range(_NR):
                    fi = cumv - (1 + r)
                    fj = plsc.load_gather(p_ref, [jnp.maximum(fi, 0)])
                    fj = jnp.where(fi >= 0, fj, 0)
                    plsc.store_scatter(ilist, [rowv, colbase + r],
                                       fj * _NH + vbase)
                    bi = cprev + r
                    bj = plsc.load_gather(p_ref, [bi])
                    bj = jnp.where(jnp.logical_and(bi < cnt, pos > 0), bj, 0)
                    plsc.store_scatter(ilist, [rowv, colbase + r + _NR],
                                       bj * _NH + vbase)

        def accum(cidx, rows, ws=ws, outbase=outbase):
            def acc_body(i, _):
                ri = rows.at[i // 8]
                m = (i % 8) * 16
                oi = outc.at[i]
                for k in range(_HD // 16):
                    a = ws[0] * ri[m, pl.ds(k * 16, 16)]
                    for r in range(1, 2 * _NR):
                        a = a + ws[r] * ri[m + r, pl.ds(k * 16, 16)]
                    oi[pl.ds(k * 16, 16)] = a
                return 0

            lax.fori_loop(0, _CH, acc_body, 0)
            pltpu.sync_copy(outc, outh.at[pl.ds(outbase + cidx * _CH, _CH)])

        def pipe(j, _, q=q):
            c0 = q * _CPU_ + 2 * j
            build(c0, il0)
            cps0 = [pltpu.async_copy(vrows.at[il0.at[jj]], rows0.at[jj], sem0)
                    for jj in range(4)]
            build(c0 + 1, il1)
            cps1 = [pltpu.async_copy(vrows.at[il1.at[jj]], rows1.at[jj], sem1)
                    for jj in range(4)]
            for cp in cps0:
                cp.wait()
            accum(c0, rows0)
            for cp in cps1:
                cp.wait()
            accum(c0 + 1, rows1)
            return 0

        lax.fori_loop(0, _CPU_ // 2, pipe, 0)


_SC_KERNEL = functools.partial(
    pl.kernel,
    out_type=jax.ShapeDtypeStruct((_BS * _NH * _L, _HD), jnp.float32),
    mesh=plsc.VectorSubcoreMesh(core_axis_name="c", subcore_axis_name="s",
                                num_cores=2, num_subcores=16),
    compiler_params=pltpu.CompilerParams(needs_layout_passes=False,
                                         use_tc_tiling_on_sc=False),
    scratch_types=[
        pltpu.VMEM((_L * _NH,), jnp.int32),
        pltpu.VMEM((_L,), jnp.int32),
        pltpu.VMEM((_L + 16,), jnp.int32),
        pltpu.VMEM((2 * _NR,), jnp.float32),
        pltpu.VMEM((4, 128), jnp.int32),
        pltpu.VMEM((4, 128), jnp.int32),
        pltpu.VMEM((4, 128, _HD), jnp.float32),
        pltpu.VMEM((4, 128, _HD), jnp.float32),
        pltpu.VMEM((_CH, _HD), jnp.float32),
        pltpu.SemaphoreType.DMA,
        pltpu.SemaphoreType.DMA,
    ],
)(_sc_body)


def kernel(hidden_states, WK, bK, WV, bV, ReadingHead, bidirection_weight):
    wkt = WK.T
    wvt = WV.T
    seg = jnp.repeat(jnp.eye(_NH, dtype=jnp.float32), _HD, axis=0)
    rh = ReadingHead.reshape(1, _NH * _HD)
    bk = bK.reshape(1, _HID)
    bv = bV.reshape(1, _HID)

    V, M = pl.pallas_call(
        _tc_body,
        grid=(_BS, _L // _LB),
        in_specs=[
            pl.BlockSpec((1, _LB, _HID), lambda b, l: (b, l, 0)),
            pl.BlockSpec((_HID, _HID), lambda b, l: (0, 0)),
            pl.BlockSpec((1, _HID), lambda b, l: (0, 0)),
            pl.BlockSpec((_HID, _HID), lambda b, l: (0, 0)),
            pl.BlockSpec((1, _HID), lambda b, l: (0, 0)),
            pl.BlockSpec((1, _HID), lambda b, l: (0, 0)),
            pl.BlockSpec((_HID, _NH), lambda b, l: (0, 0)),
        ],
        out_specs=[
            pl.BlockSpec((1, _LB, _HID), lambda b, l: (b, l, 0)),
            pl.BlockSpec((1, _LB, _NH), lambda b, l: (b, l, 0)),
        ],
        out_shape=[
            jax.ShapeDtypeStruct((_BS, _L, _HID), jnp.float32),
            jax.ShapeDtypeStruct((_BS, _L, _NH), jnp.int32),
        ],
    )(hidden_states, wkt, bk, wvt, bv, rh, seg)

    vrows = V.reshape(_BS * _L * _NH, _HD)
    mask2 = M.reshape(_BS, _L * _NH)
    wmat = bidirection_weight.reshape(_NH, 2 * _NR)
    outr = _SC_KERNEL(vrows, mask2, wmat)
    return (outr.reshape(_BS, _NH, _L, _HD)
            .transpose(0, 2, 1, 3)
            .reshape(_BS, _L, _NH * _HD))


# speedup vs baseline: 2639.9833x; 1.0540x over previous
import functools

import jax
import jax.numpy as jnp
from jax import lax
from jax.experimental import pallas as pl
from jax.experimental.pallas import tpu as pltpu
from jax.experimental.pallas import tpu_sc as plsc

_NH = 12
_HD = 64
_HID = 768
_NR = 8
_L = 2048
_BS = 2
_LB = 512
_CH = 32
_NCHUNK = _L // _CH
_NPAIR = _BS * _NH


def _tc_body(h_ref, wkt_ref, bk_ref, wvt_ref, bv_ref, rh_ref, seg_ref,
             v_ref, m_ref):
    x = h_ref[0]
    v = jnp.dot(x, wvt_ref[...], preferred_element_type=jnp.float32,
                precision=lax.Precision.DEFAULT) + bv_ref[0]
    v_ref[0] = jnp.maximum(v, 0.0)
    k = jnp.dot(x, wkt_ref[...], preferred_element_type=jnp.float32,
                precision=lax.Precision.DEFAULT) + bk_ref[0]
    k = jnp.maximum(k, 0.0)
    k16 = k.astype(jnp.bfloat16).astype(jnp.float32)
    rh16 = rh_ref[0].astype(jnp.bfloat16).astype(jnp.float32)
    dot = jnp.dot(k16 * rh16, seg_ref[...],
                  preferred_element_type=jnp.float32,
                  precision=lax.Precision.HIGHEST)
    m_ref[0] = (dot > 0.5).astype(jnp.int32)


_UPT = 3
_CPU_ = _NCHUNK // 4


def _sc_body(vrows, maskh, wh, outh, maskblk, cum, p_ref, wrow,
             il0, il1, rows0, rows1, outc, oidx, sem0, sem1, osem):
    wid = lax.axis_index("s") * 2 + lax.axis_index("c")
    lanes = lax.iota(jnp.int32, 16)
    subrow = jnp.where(lanes >= 8, 1, 0)
    colbase = (lanes % 8) * 16

    def unit_body(s, _):
        u = wid * _UPT + s
        p = u // 4
        q = u % 4
        b = p // _NH
        h = p % _NH
        pltpu.sync_copy(maskh.at[b], maskblk)
        pltpu.sync_copy(wh.at[h], wrow)

        def scan_step(v, c, h=h):
            base = v * 16
            pos = lanes + base
            mvec = plsc.load_gather(maskblk, [pos * _NH + h])
            mvec = jnp.where(pos == 0, 0, mvec)
            cum[pl.ds(base, 16)] = plsc.cumsum(mvec) + c
            plsc.store_scatter(p_ref, [jnp.maximum(cum[pl.ds(base, 16)] - 1, 0)],
                               pos, mask=mvec != 0)
            return c + jnp.sum(mvec)

        cnt = lax.fori_loop(0, _L // 16, scan_step, jnp.int32(0))

        wvec = wrow[...]
        ws = [wvec[r] for r in range(2 * _NR)]
        vbase = (b * _L) * _NH + h
        outbase = (b * _NH + h) * _L

        def build(cidx, ilist, cnt=cnt, vbase=vbase):
            for v in range(_CH // 16):
                pos = lanes + cidx * _CH + v * 16
                cumv = cum[pl.ds(cidx * _CH + v * 16, 16)]
                cprev = plsc.load_gather(cum, [jnp.maximum(pos - 1, 0)])
                cprev = jnp.where(pos > 0, cprev, 0)
                rowv = subrow + 2 * v
                for r in range(_NR):
                    fi = cumv - (1 + r)
                    fj = plsc.load_gather(p_ref, [jnp.maximum(fi, 0)])
                    fj = jnp.where(fi >= 0, fj, 0)
                    plsc.store_scatter(ilist, [rowv, colbase + r],
                                       fj * _NH + vbase)
                    bi = cprev + r
                    bj = plsc.load_gather(p_ref, [bi])
                    bj = jnp.where(jnp.logical_and(bi < cnt, pos > 0), bj, 0)
                    plsc.store_scatter(ilist, [rowv, colbase + r + _NR],
                                       bj * _NH + vbase)

        def accum(cidx, rows, oidx, osem, ws=ws, b=b, h=h):
            def acc_body(j, _):
                ri = rows.at[j]
                for m in range(8):
                    oi = outc.at[j * 8 + m]
                    for k in range(_HD // 16):
                        a = ws[0] * ri[m * 16, pl.ds(k * 16, 16)]
                        for r in range(1, 2 * _NR):
                            a = a + ws[r] * ri[m * 16 + r, pl.ds(k * 16, 16)]
                        oi[pl.ds(k * 16, 16)] = a
                return 0

            lax.fori_loop(0, 4, acc_body, 0)
            for v in range(_CH // 16):
                orow = (b * _L + cidx * _CH + v * 16 + lanes) * _NH + h
                oidx[pl.ds(v * 16, 16)] = orow
            pltpu.async_copy(outc, outh.at[oidx], osem).wait()

        def pipe(j, _, q=q):
            c0 = q * _CPU_ + 2 * j
            build(c0, il0)
            cps0 = [pltpu.async_copy(vrows.at[il0.at[jj]], rows0.at[jj], sem0)
                    for jj in range(4)]
            build(c0 + 1, il1)
            cps1 = [pltpu.async_copy(vrows.at[il1.at[jj]], rows1.at[jj], sem1)
                    for jj in range(4)]
            for cp in cps0:
                cp.wait()
            accum(c0, rows0, oidx, osem)
            for cp in cps1:
                cp.wait()
            accum(c0 + 1, rows1, oidx, osem)
            return 0

        lax.fori_loop(0, _CPU_ // 2, pipe, 0)
        return 0

    lax.fori_loop(0, _UPT, unit_body, 0)


_SC_KERNEL = functools.partial(
    pl.kernel,
    out_type=jax.ShapeDtypeStruct((_BS * _NH * _L, _HD), jnp.float32),
    mesh=plsc.VectorSubcoreMesh(core_axis_name="c", subcore_axis_name="s",
                                num_cores=2, num_subcores=16),
    compiler_params=pltpu.CompilerParams(needs_layout_passes=False,
                                         use_tc_tiling_on_sc=False),
    scratch_types=[
        pltpu.VMEM((_L * _NH,), jnp.int32),
        pltpu.VMEM((_L,), jnp.int32),
        pltpu.VMEM((_L + 16,), jnp.int32),
        pltpu.VMEM((2 * _NR,), jnp.float32),
        pltpu.VMEM((4, 128), jnp.int32),
        pltpu.VMEM((4, 128), jnp.int32),
        pltpu.VMEM((4, 128, _HD), jnp.float32),
        pltpu.VMEM((4, 128, _HD), jnp.float32),
        pltpu.VMEM((_CH, _HD), jnp.float32),
        pltpu.VMEM((_CH,), jnp.int32),
        pltpu.SemaphoreType.DMA,
        pltpu.SemaphoreType.DMA,
        pltpu.SemaphoreType.DMA,
    ],
)(_sc_body)


def kernel(hidden_states, WK, bK, WV, bV, ReadingHead, bidirection_weight):
    wkt = WK.T
    wvt = WV.T
    seg = jnp.repeat(jnp.eye(_NH, dtype=jnp.float32), _HD, axis=0)
    rh = ReadingHead.reshape(1, _NH * _HD)
    bk = bK.reshape(1, _HID)
    bv = bV.reshape(1, _HID)

    V, M = pl.pallas_call(
        _tc_body,
        grid=(_BS, _L // _LB),
        in_specs=[
            pl.BlockSpec((1, _LB, _HID), lambda b, l: (b, l, 0)),
            pl.BlockSpec((_HID, _HID), lambda b, l: (0, 0)),
            pl.BlockSpec((1, _HID), lambda b, l: (0, 0)),
            pl.BlockSpec((_HID, _HID), lambda b, l: (0, 0)),
            pl.BlockSpec((1, _HID), lambda b, l: (0, 0)),
            pl.BlockSpec((1, _HID), lambda b, l: (0, 0)),
            pl.BlockSpec((_HID, _NH), lambda b, l: (0, 0)),
        ],
        out_specs=[
            pl.BlockSpec((1, _LB, _HID), lambda b, l: (b, l, 0)),
            pl.BlockSpec((1, _LB, _NH), lambda b, l: (b, l, 0)),
        ],
        out_shape=[
            jax.ShapeDtypeStruct((_BS, _L, _HID), jnp.float32),
            jax.ShapeDtypeStruct((_BS, _L, _NH), jnp.int32),
        ],
    )(hidden_states, wkt, bk, wvt, bv, rh, seg)

    vrows = V.reshape(_BS * _L * _NH, _HD)
    mask2 = M.reshape(_BS, _L * _NH)
    wmat = bidirection_weight.reshape(_NH, 2 * _NR)
    outr = _SC_KERNEL(vrows, mask2, wmat)
    return outr.reshape(_BS, _L, _NH * _HD)


# speedup vs baseline: 3557.9174x; 1.3477x over previous
import functools

import jax
import jax.numpy as jnp
from jax import lax
from jax.experimental import pallas as pl
from jax.experimental.pallas import tpu as pltpu
from jax.experimental.pallas import tpu_sc as plsc

_NH = 12
_HD = 64
_HID = 768
_NR = 8
_L = 2048
_BS = 2
_LB = 512
_CH = 32
_NCHUNK = _L // _CH
_NPAIR = _BS * _NH


def _tc_body(h_ref, wkt_ref, bk_ref, wvt_ref, bv_ref, rh_ref, seg_ref,
             v_ref, m_ref):
    x = h_ref[0]
    v = jnp.dot(x, wvt_ref[...], preferred_element_type=jnp.float32,
                precision=lax.Precision.DEFAULT) + bv_ref[0]
    v_ref[0] = jnp.maximum(v, 0.0)
    k = jnp.dot(x, wkt_ref[...], preferred_element_type=jnp.float32,
                precision=lax.Precision.DEFAULT) + bk_ref[0]
    k = jnp.maximum(k, 0.0)
    k16 = k.astype(jnp.bfloat16).astype(jnp.float32)
    rh16 = rh_ref[0].astype(jnp.bfloat16).astype(jnp.float32)
    dot = jnp.dot(k16 * rh16, seg_ref[...],
                  preferred_element_type=jnp.float32,
                  precision=lax.Precision.HIGHEST)
    m_ref[0] = (dot > 0.5).astype(jnp.int32)


_UPT = 3
_CPU_ = _NCHUNK // 4


_W = _CH + 16


def _sc_body(vrows, maskh, wh, outh, maskblk, cum, p_ref, wrow,
             il0, il1, rows0, rows1, outc, oidx, sem0, sem1, osem):
    wid = lax.axis_index("s") * 2 + lax.axis_index("c")
    lanes = lax.iota(jnp.int32, 16)

    def unit_body(s, _):
        u = wid * _UPT + s
        p = u // 4
        q = u % 4
        b = p // _NH
        h = p % _NH
        pltpu.sync_copy(maskh.at[b], maskblk)
        pltpu.sync_copy(wh.at[h], wrow)

        def scan_step(v, c, h=h):
            base = v * 16
            pos = lanes + base
            mvec = plsc.load_gather(maskblk, [pos * _NH + h])
            mvec = jnp.where(pos == 0, 0, mvec)
            cum[pl.ds(base, 16)] = plsc.cumsum(mvec) + c
            plsc.store_scatter(p_ref, [jnp.maximum(cum[pl.ds(base, 16)] - 1, 0)],
                               pos, mask=mvec != 0)
            return c + jnp.sum(mvec)

        cnt = lax.fori_loop(0, _L // 16, scan_step, jnp.int32(0))

        wvec = wrow[...]
        ws = [wvec[r] for r in range(2 * _NR)]
        vbase = (b * _L) * _NH + h

        def build(cidx, ilist, cnt=cnt, vbase=vbase):
            start = cidx * _CH
            w0 = cum[pl.ds(start, 16)][0] - 8
            for v in range(_W // 16):
                pj = w0 + lanes + v * 16
                valid = jnp.logical_and(pj >= 0, pj < cnt)
                pjc = jnp.minimum(jnp.maximum(pj, 0), _L + 15)
                posj = plsc.load_gather(p_ref, [pjc])
                posj = jnp.where(valid, posj, 0)
                ilist[pl.ds(v * 16, 16)] = posj * _NH + vbase
            return w0

        def accum(cidx, rows, w0, ws=ws, b=b, h=h):
            start = cidx * _CH

            def acc_group(j8, _):
                base = start + j8 * 8
                cumv = cum[pl.ds(base, 16)]
                pos = base + lanes
                cpv = plsc.load_gather(cum, [jnp.maximum(pos - 1, 0)])
                lf = cumv - 1 - w0
                lb = jnp.where(pos > 0, cpv - w0, 0)
                for m in range(8):
                    lfm = lf[m]
                    lbm = lb[m]
                    oi = outc.at[j8 * 8 + m]
                    for k in range(_HD // 16):
                        a = ws[0] * rows[lfm, pl.ds(k * 16, 16)]
                        for r in range(1, _NR):
                            a = a + ws[r] * rows[lfm - r, pl.ds(k * 16, 16)]
                        for r in range(_NR):
                            a = a + ws[_NR + r] * rows[lbm + r,
                                                       pl.ds(k * 16, 16)]
                        oi[pl.ds(k * 16, 16)] = a
                return 0

            lax.fori_loop(0, _CH // 8, acc_group, 0)
            for v in range(_CH // 16):
                orow = (b * _L + cidx * _CH + v * 16 + lanes) * _NH + h
                oidx[pl.ds(v * 16, 16)] = orow
            pltpu.async_copy(outc, outh.at[oidx], osem).wait()

        def pipe(j, _, q=q):
            c0 = q * _CPU_ + 2 * j
            w00 = build(c0, il0)
            cp0 = pltpu.async_copy(vrows.at[il0], rows0, sem0)
            w01 = build(c0 + 1, il1)
            cp1 = pltpu.async_copy(vrows.at[il1], rows1, sem1)
            cp0.wait()
            accum(c0, rows0, w00)
            cp1.wait()
            accum(c0 + 1, rows1, w01)
            return 0

        lax.fori_loop(0, _CPU_ // 2, pipe, 0)
        return 0

    lax.fori_loop(0, _UPT, unit_body, 0)


_SC_KERNEL = functools.partial(
    pl.kernel,
    out_type=jax.ShapeDtypeStruct((_BS * _NH * _L, _HD), jnp.float32),
    mesh=plsc.VectorSubcoreMesh(core_axis_name="c", subcore_axis_name="s",
                                num_cores=2, num_subcores=16),
    compiler_params=pltpu.CompilerParams(needs_layout_passes=False,
                                         use_tc_tiling_on_sc=False),
    scratch_types=[
        pltpu.VMEM((_L * _NH,), jnp.int32),
        pltpu.VMEM((_L,), jnp.int32),
        pltpu.VMEM((_L + 16,), jnp.int32),
        pltpu.VMEM((2 * _NR,), jnp.float32),
        pltpu.VMEM((_W,), jnp.int32),
        pltpu.VMEM((_W,), jnp.int32),
        pltpu.VMEM((_W, _HD), jnp.float32),
        pltpu.VMEM((_W, _HD), jnp.float32),
        pltpu.VMEM((_CH, _HD), jnp.float32),
        pltpu.VMEM((_CH,), jnp.int32),
        pltpu.SemaphoreType.DMA,
        pltpu.SemaphoreType.DMA,
        pltpu.SemaphoreType.DMA,
    ],
)(_sc_body)


def kernel(hidden_states, WK, bK, WV, bV, ReadingHead, bidirection_weight):
    wkt = WK.T
    wvt = WV.T
    seg = jnp.repeat(jnp.eye(_NH, dtype=jnp.float32), _HD, axis=0)
    rh = ReadingHead.reshape(1, _NH * _HD)
    bk = bK.reshape(1, _HID)
    bv = bV.reshape(1, _HID)

    V, M = pl.pallas_call(
        _tc_body,
        grid=(_BS, _L // _LB),
        in_specs=[
            pl.BlockSpec((1, _LB, _HID), lambda b, l: (b, l, 0)),
            pl.BlockSpec((_HID, _HID), lambda b, l: (0, 0)),
            pl.BlockSpec((1, _HID), lambda b, l: (0, 0)),
            pl.BlockSpec((_HID, _HID), lambda b, l: (0, 0)),
            pl.BlockSpec((1, _HID), lambda b, l: (0, 0)),
            pl.BlockSpec((1, _HID), lambda b, l: (0, 0)),
            pl.BlockSpec((_HID, _NH), lambda b, l: (0, 0)),
        ],
        out_specs=[
            pl.BlockSpec((1, _LB, _HID), lambda b, l: (b, l, 0)),
            pl.BlockSpec((1, _LB, _NH), lambda b, l: (b, l, 0)),
        ],
        out_shape=[
            jax.ShapeDtypeStruct((_BS, _L, _HID), jnp.float32),
            jax.ShapeDtypeStruct((_BS, _L, _NH), jnp.int32),
        ],
    )(hidden_states, wkt, bk, wvt, bv, rh, seg)

    vrows = V.reshape(_BS * _L * _NH, _HD)
    mask2 = M.reshape(_BS, _L * _NH)
    wmat = bidirection_weight.reshape(_NH, 2 * _NR)
    outr = _SC_KERNEL(vrows, mask2, wmat)
    return outr.reshape(_BS, _L, _NH * _HD)
